# initial kernel scaffold (unmeasured)
import jax
import jax.numpy as jnp
from jax import lax
from jax.experimental import pallas as pl
from jax.experimental.pallas import tpu as pltpu

T = 1024
T_LOC = 512
D = 1024
F = 2048
E = 8
E_LOC = 4
CAP = 384


def _exchange_body(x_ref, r_ref, xf_ref, rf_ref, send_sems, recv_sems):
    ix = lax.axis_index("x")
    iy = lax.axis_index("y")
    iz = lax.axis_index("z")
    partner = (1 - ix, iy, iz)

    barrier = pltpu.get_barrier_semaphore()
    pl.semaphore_signal(
        barrier, inc=1, device_id=partner, device_id_type=pl.DeviceIdType.MESH
    )
    pl.semaphore_wait(barrier, 1)

    xf_ref[pl.ds(ix * T_LOC, T_LOC), :] = x_ref[:, :]
    rf_ref[ix] = r_ref[:, :]

    rdma_x = pltpu.make_async_remote_copy(
        src_ref=x_ref,
        dst_ref=xf_ref.at[pl.ds(ix * T_LOC, T_LOC)],
        send_sem=send_sems.at[0],
        recv_sem=recv_sems.at[0],
        device_id=partner,
        device_id_type=pl.DeviceIdType.MESH,
    )
    rdma_r = pltpu.make_async_remote_copy(
        src_ref=r_ref,
        dst_ref=rf_ref.at[ix],
        send_sem=send_sems.at[1],
        recv_sem=recv_sems.at[1],
        device_id=partner,
        device_id_type=pl.DeviceIdType.MESH,
    )
    rdma_x.start()
    rdma_r.start()
    rdma_x.wait()
    rdma_r.wait()


def _ffn_body(xsel_ref, w1_ref, w2_ref, wsel_ref, out_ref):
    h = jnp.maximum(
        jnp.dot(xsel_ref[0], w1_ref[0], preferred_element_type=jnp.float32), 0.0
    )
    out = jnp.dot(h, w2_ref[0], preferred_element_type=jnp.float32)
    out_ref[0] = out * wsel_ref[0][:, None]


def _combine_body(of_ref, out_ref, comm_ref, send_sem, recv_sem):
    ix = lax.axis_index("x")
    iy = lax.axis_index("y")
    iz = lax.axis_index("z")
    partner = (1 - ix, iy, iz)

    barrier = pltpu.get_barrier_semaphore()
    pl.semaphore_signal(
        barrier, inc=1, device_id=partner, device_id_type=pl.DeviceIdType.MESH
    )
    pl.semaphore_wait(barrier, 1)

    pr = 1 - ix
    rdma = pltpu.make_async_remote_copy(
        src_ref=of_ref.at[pl.ds(pr * T_LOC, T_LOC)],
        dst_ref=comm_ref,
        send_sem=send_sem,
        recv_sem=recv_sem,
        device_id=partner,
        device_id_type=pl.DeviceIdType.MESH,
    )
    rdma.start()
    rdma.wait()

    out_ref[:, :] = of_ref[pl.ds(ix * T_LOC, T_LOC), :] + comm_ref[:, :]


def kernel(x, router, W1, W2):
    ix = lax.axis_index("x")

    xf, rf = pl.pallas_call(
        _exchange_body,
        out_shape=[
            jax.ShapeDtypeStruct((T, D), jnp.float32),
            jax.ShapeDtypeStruct((2, T, E_LOC), jnp.float32),
        ],
        in_specs=[
            pl.BlockSpec(memory_space=pltpu.VMEM),
            pl.BlockSpec(memory_space=pltpu.VMEM),
        ],
        out_specs=[
            pl.BlockSpec(memory_space=pltpu.VMEM),
            pl.BlockSpec(memory_space=pltpu.VMEM),
        ],
        scratch_shapes=[
            pltpu.SemaphoreType.DMA((2,)),
            pltpu.SemaphoreType.DMA((2,)),
        ],
        compiler_params=pltpu.CompilerParams(collective_id=0),
    )(x, router)

    router_full = jnp.concatenate([rf[0], rf[1]], axis=1)

    gates = jnp.dot(xf, router_full, precision=lax.Precision.HIGHEST)
    topv, topi = lax.top_k(gates, 2)
    w = jax.nn.softmax(topv, axis=1)

    e_global = ix * E_LOC + jnp.arange(E_LOC)
    w_e = jnp.sum(
        jnp.where(topi[:, :, None] == e_global[None, None, :], w[:, :, None], 0.0),
        axis=1,
    )
    selected = w_e > 0.0

    order = jnp.argsort(jnp.where(selected, 0, 1), axis=0, stable=True)
    idx = order[:CAP]
    idx_t = idx.T
    xsel = xf[idx_t]
    wsel = jnp.take_along_axis(w_e, idx, axis=0).T

    out_sel = pl.pallas_call(
        _ffn_body,
        grid=(E_LOC,),
        out_shape=jax.ShapeDtypeStruct((E_LOC, CAP, D), jnp.float32),
        in_specs=[
            pl.BlockSpec((1, CAP, D), lambda e: (e, 0, 0)),
            pl.BlockSpec((1, D, F), lambda e: (e, 0, 0)),
            pl.BlockSpec((1, F, D), lambda e: (e, 0, 0)),
            pl.BlockSpec((1, CAP), lambda e: (e, 0)),
        ],
        out_specs=pl.BlockSpec((1, CAP, D), lambda e: (e, 0, 0)),
    )(xsel, W1, W2, wsel)

    of = jnp.zeros((T, D), jnp.float32).at[idx_t].add(out_sel)

    out = pl.pallas_call(
        _combine_body,
        out_shape=jax.ShapeDtypeStruct((T_LOC, D), jnp.float32),
        in_specs=[pl.BlockSpec(memory_space=pltpu.VMEM)],
        out_specs=pl.BlockSpec(memory_space=pltpu.VMEM),
        scratch_shapes=[
            pltpu.VMEM((T_LOC, D), jnp.float32),
            pltpu.SemaphoreType.DMA,
            pltpu.SemaphoreType.DMA,
        ],
        compiler_params=pltpu.CompilerParams(collective_id=1),
    )(of)

    return out


# baseline (device time: 194573 ns/iter reference)
import jax
import jax.numpy as jnp
from jax import lax
from jax.experimental import pallas as pl
from jax.experimental.pallas import tpu as pltpu

T = 1024
T_LOC = 512
D = 1024
F = 2048
E = 8
E_LOC = 4
CAP = 384


def _exchange_body(x_ref, r_ref, xf_ref, rf_ref, send_sems, recv_sems):
    ix = lax.axis_index("x")
    iy = lax.axis_index("y")
    iz = lax.axis_index("z")
    partner = (1 - ix, iy, iz)

    barrier = pltpu.get_barrier_semaphore()
    pl.semaphore_signal(
        barrier, inc=1, device_id=partner, device_id_type=pl.DeviceIdType.MESH
    )
    pl.semaphore_wait(barrier, 1)

    xf_ref[pl.ds(ix * T_LOC, T_LOC), :] = x_ref[:, :]
    rf_ref[ix] = r_ref[:, :]

    rdma_x = pltpu.make_async_remote_copy(
        src_ref=x_ref,
        dst_ref=xf_ref.at[pl.ds(ix * T_LOC, T_LOC)],
        send_sem=send_sems.at[0],
        recv_sem=recv_sems.at[0],
        device_id=partner,
        device_id_type=pl.DeviceIdType.MESH,
    )
    rdma_r = pltpu.make_async_remote_copy(
        src_ref=r_ref,
        dst_ref=rf_ref.at[ix],
        send_sem=send_sems.at[1],
        recv_sem=recv_sems.at[1],
        device_id=partner,
        device_id_type=pl.DeviceIdType.MESH,
    )
    rdma_x.start()
    rdma_r.start()
    rdma_x.wait()
    rdma_r.wait()


def _ffn_body(xsel_ref, w1_ref, w2_ref, wsel_ref, out_ref):
    h = jnp.maximum(
        jnp.dot(xsel_ref[0], w1_ref[0], preferred_element_type=jnp.float32), 0.0
    )
    out = jnp.dot(h, w2_ref[0], preferred_element_type=jnp.float32)
    out_ref[0] = out * wsel_ref[0]


def _combine_body(of_ref, out_ref, comm_ref, send_sem, recv_sem):
    ix = lax.axis_index("x")
    iy = lax.axis_index("y")
    iz = lax.axis_index("z")
    partner = (1 - ix, iy, iz)

    barrier = pltpu.get_barrier_semaphore()
    pl.semaphore_signal(
        barrier, inc=1, device_id=partner, device_id_type=pl.DeviceIdType.MESH
    )
    pl.semaphore_wait(barrier, 1)

    pr = 1 - ix
    rdma = pltpu.make_async_remote_copy(
        src_ref=of_ref.at[pl.ds(pr * T_LOC, T_LOC)],
        dst_ref=comm_ref,
        send_sem=send_sem,
        recv_sem=recv_sem,
        device_id=partner,
        device_id_type=pl.DeviceIdType.MESH,
    )
    rdma.start()
    rdma.wait()

    out_ref[:, :] = of_ref[pl.ds(ix * T_LOC, T_LOC), :] + comm_ref[:, :]


def kernel(x, router, W1, W2):
    ix = lax.axis_index("x")

    xf, rf = pl.pallas_call(
        _exchange_body,
        out_shape=[
            jax.ShapeDtypeStruct((T, D), jnp.float32),
            jax.ShapeDtypeStruct((2, T, E_LOC), jnp.float32),
        ],
        in_specs=[
            pl.BlockSpec(memory_space=pltpu.VMEM),
            pl.BlockSpec(memory_space=pltpu.VMEM),
        ],
        out_specs=[
            pl.BlockSpec(memory_space=pltpu.VMEM),
            pl.BlockSpec(memory_space=pltpu.VMEM),
        ],
        scratch_shapes=[
            pltpu.SemaphoreType.DMA((2,)),
            pltpu.SemaphoreType.DMA((2,)),
        ],
        compiler_params=pltpu.CompilerParams(collective_id=0),
    )(x, router)

    router_full = jnp.concatenate([rf[0], rf[1]], axis=1)

    gates = jnp.dot(xf, router_full, precision=lax.Precision.HIGHEST)
    topv, topi = lax.top_k(gates, 2)
    w = jax.nn.softmax(topv, axis=1)

    e_global = ix * E_LOC + jnp.arange(E_LOC)
    w_e = jnp.sum(
        jnp.where(topi[:, :, None] == e_global[None, None, :], w[:, :, None], 0.0),
        axis=1,
    )
    selected = w_e > 0.0

    order = jnp.argsort(jnp.where(selected, 0, 1), axis=0, stable=True)
    idx = order[:CAP]
    idx_t = idx.T
    xsel = xf[idx_t]
    wsel = jnp.take_along_axis(w_e, idx, axis=0).T[:, :, None]

    out_sel = pl.pallas_call(
        _ffn_body,
        grid=(E_LOC,),
        out_shape=jax.ShapeDtypeStruct((E_LOC, CAP, D), jnp.float32),
        in_specs=[
            pl.BlockSpec((1, CAP, D), lambda e: (e, 0, 0)),
            pl.BlockSpec((1, D, F), lambda e: (e, 0, 0)),
            pl.BlockSpec((1, F, D), lambda e: (e, 0, 0)),
            pl.BlockSpec((1, CAP, 1), lambda e: (e, 0, 0)),
        ],
        out_specs=pl.BlockSpec((1, CAP, D), lambda e: (e, 0, 0)),
        compiler_params=pltpu.CompilerParams(vmem_limit_bytes=100 * 1024 * 1024),
    )(xsel, W1, W2, wsel)

    of = jnp.zeros((T, D), jnp.float32).at[idx_t].add(out_sel)

    out = pl.pallas_call(
        _combine_body,
        out_shape=jax.ShapeDtypeStruct((T_LOC, D), jnp.float32),
        in_specs=[pl.BlockSpec(memory_space=pltpu.VMEM)],
        out_specs=pl.BlockSpec(memory_space=pltpu.VMEM),
        scratch_shapes=[
            pltpu.VMEM((T_LOC, D), jnp.float32),
            pltpu.SemaphoreType.DMA,
            pltpu.SemaphoreType.DMA,
        ],
        compiler_params=pltpu.CompilerParams(collective_id=1),
    )(of)

    return out


# device time: 117502 ns/iter; 1.6559x vs baseline; 1.6559x over previous
import jax
import jax.numpy as jnp
from jax import lax
from jax.experimental import pallas as pl
from jax.experimental.pallas import tpu as pltpu

T = 1024
T_LOC = 512
D = 1024
F = 2048
E = 8
E_LOC = 4
CAP = 384


def _exchange_body(x_ref, r_ref, xf_ref, rf_ref, send_sems, recv_sems):
    ix = lax.axis_index("x")
    iy = lax.axis_index("y")
    iz = lax.axis_index("z")
    partner = (1 - ix, iy, iz)

    barrier = pltpu.get_barrier_semaphore()
    pl.semaphore_signal(
        barrier, inc=1, device_id=partner, device_id_type=pl.DeviceIdType.MESH
    )
    pl.semaphore_wait(barrier, 1)

    xf_ref[pl.ds(ix * T_LOC, T_LOC), :] = x_ref[:, :]
    rf_ref[ix] = r_ref[:, :]

    rdma_x = pltpu.make_async_remote_copy(
        src_ref=x_ref,
        dst_ref=xf_ref.at[pl.ds(ix * T_LOC, T_LOC)],
        send_sem=send_sems.at[0],
        recv_sem=recv_sems.at[0],
        device_id=partner,
        device_id_type=pl.DeviceIdType.MESH,
    )
    rdma_r = pltpu.make_async_remote_copy(
        src_ref=r_ref,
        dst_ref=rf_ref.at[ix],
        send_sem=send_sems.at[1],
        recv_sem=recv_sems.at[1],
        device_id=partner,
        device_id_type=pl.DeviceIdType.MESH,
    )
    rdma_x.start()
    rdma_r.start()
    rdma_x.wait()
    rdma_r.wait()


def _ffn_body(xf_ref, idx_ref, wsel_ref, w1_ref, w2_ref, out_ref):
    e = pl.program_id(0)
    idx_col = idx_ref[0]
    p = (
        lax.broadcasted_iota(jnp.int32, (CAP, T), 1) == idx_col
    ).astype(jnp.float32)
    xsel = jnp.dot(p, xf_ref[:, :], preferred_element_type=jnp.float32)
    h = jnp.maximum(
        jnp.dot(xsel, w1_ref[0], preferred_element_type=jnp.float32), 0.0
    )
    o = jnp.dot(h, w2_ref[0], preferred_element_type=jnp.float32) * wsel_ref[0]
    contrib = lax.dot_general(
        p, o, (((0,), (0,)), ((), ())), preferred_element_type=jnp.float32
    )

    @pl.when(e == 0)
    def _():
        out_ref[:, :] = contrib

    @pl.when(e != 0)
    def _():
        out_ref[:, :] = out_ref[:, :] + contrib


def _combine_body(of_ref, out_ref, comm_ref, send_sem, recv_sem):
    ix = lax.axis_index("x")
    iy = lax.axis_index("y")
    iz = lax.axis_index("z")
    partner = (1 - ix, iy, iz)

    barrier = pltpu.get_barrier_semaphore()
    pl.semaphore_signal(
        barrier, inc=1, device_id=partner, device_id_type=pl.DeviceIdType.MESH
    )
    pl.semaphore_wait(barrier, 1)

    pr = 1 - ix
    rdma = pltpu.make_async_remote_copy(
        src_ref=of_ref.at[pl.ds(pr * T_LOC, T_LOC)],
        dst_ref=comm_ref,
        send_sem=send_sem,
        recv_sem=recv_sem,
        device_id=partner,
        device_id_type=pl.DeviceIdType.MESH,
    )
    rdma.start()
    rdma.wait()

    out_ref[:, :] = of_ref[pl.ds(ix * T_LOC, T_LOC), :] + comm_ref[:, :]


def kernel(x, router, W1, W2):
    ix = lax.axis_index("x")

    xf, rf = pl.pallas_call(
        _exchange_body,
        out_shape=[
            jax.ShapeDtypeStruct((T, D), jnp.float32),
            jax.ShapeDtypeStruct((2, T, E_LOC), jnp.float32),
        ],
        in_specs=[
            pl.BlockSpec(memory_space=pltpu.VMEM),
            pl.BlockSpec(memory_space=pltpu.VMEM),
        ],
        out_specs=[
            pl.BlockSpec(memory_space=pltpu.VMEM),
            pl.BlockSpec(memory_space=pltpu.VMEM),
        ],
        scratch_shapes=[
            pltpu.SemaphoreType.DMA((2,)),
            pltpu.SemaphoreType.DMA((2,)),
        ],
        compiler_params=pltpu.CompilerParams(collective_id=0),
    )(x, router)

    router_full = jnp.concatenate([rf[0], rf[1]], axis=1)

    gates = jnp.dot(xf, router_full, precision=lax.Precision.HIGHEST)
    topv, topi = lax.top_k(gates, 2)
    w = jax.nn.softmax(topv, axis=1)

    e_global = ix * E_LOC + jnp.arange(E_LOC)
    w_e = jnp.sum(
        jnp.where(topi[:, :, None] == e_global[None, None, :], w[:, :, None], 0.0),
        axis=1,
    )
    selected = w_e > 0.0

    order = jnp.argsort(jnp.where(selected, 0, 1), axis=0, stable=True)
    idx = order[:CAP]
    idx_in = idx.T[:, :, None]
    wsel = jnp.take_along_axis(w_e, idx, axis=0).T[:, :, None]

    of = pl.pallas_call(
        _ffn_body,
        grid=(E_LOC,),
        out_shape=jax.ShapeDtypeStruct((T, D), jnp.float32),
        in_specs=[
            pl.BlockSpec((T, D), lambda e: (0, 0)),
            pl.BlockSpec((1, CAP, 1), lambda e: (e, 0, 0)),
            pl.BlockSpec((1, CAP, 1), lambda e: (e, 0, 0)),
            pl.BlockSpec((1, D, F), lambda e: (e, 0, 0)),
            pl.BlockSpec((1, F, D), lambda e: (e, 0, 0)),
        ],
        out_specs=pl.BlockSpec((T, D), lambda e: (0, 0)),
        compiler_params=pltpu.CompilerParams(vmem_limit_bytes=100 * 1024 * 1024),
    )(xf, idx_in, wsel, W1, W2)

    out = pl.pallas_call(
        _combine_body,
        out_shape=jax.ShapeDtypeStruct((T_LOC, D), jnp.float32),
        in_specs=[pl.BlockSpec(memory_space=pltpu.VMEM)],
        out_specs=pl.BlockSpec(memory_space=pltpu.VMEM),
        scratch_shapes=[
            pltpu.VMEM((T_LOC, D), jnp.float32),
            pltpu.SemaphoreType.DMA,
            pltpu.SemaphoreType.DMA,
        ],
        compiler_params=pltpu.CompilerParams(collective_id=1),
    )(of)

    return out


# device time: 115546 ns/iter; 1.6839x vs baseline; 1.0169x over previous
import jax
import jax.numpy as jnp
from jax import lax
from jax.experimental import pallas as pl
from jax.experimental.pallas import tpu as pltpu

T = 1024
T_LOC = 512
D = 1024
F = 2048
E = 8
E_LOC = 4
CAP = 384


def _exchange_body(x_ref, r_ref, xf_ref, w_ref, rsc_ref, send_sems, recv_sems):
    ix = lax.axis_index("x")
    iy = lax.axis_index("y")
    iz = lax.axis_index("z")
    partner = (1 - ix, iy, iz)

    barrier = pltpu.get_barrier_semaphore()
    pl.semaphore_signal(
        barrier, inc=1, device_id=partner, device_id_type=pl.DeviceIdType.MESH
    )
    pl.semaphore_wait(barrier, 1)

    xf_ref[pl.ds(ix * T_LOC, T_LOC), :] = x_ref[:, :]
    rsc_ref[ix] = r_ref[:, :]

    rdma_x = pltpu.make_async_remote_copy(
        src_ref=x_ref,
        dst_ref=xf_ref.at[pl.ds(ix * T_LOC, T_LOC)],
        send_sem=send_sems.at[0],
        recv_sem=recv_sems.at[0],
        device_id=partner,
        device_id_type=pl.DeviceIdType.MESH,
    )
    rdma_r = pltpu.make_async_remote_copy(
        src_ref=r_ref,
        dst_ref=rsc_ref.at[ix],
        send_sem=send_sems.at[1],
        recv_sem=recv_sems.at[1],
        device_id=partner,
        device_id_type=pl.DeviceIdType.MESH,
    )
    rdma_x.start()
    rdma_r.start()
    rdma_x.wait()
    rdma_r.wait()

    g = jnp.concatenate(
        [
            jnp.dot(xf_ref[:, :], rsc_ref[0], preferred_element_type=jnp.float32,
                    precision=lax.Precision.HIGHEST),
            jnp.dot(xf_ref[:, :], rsc_ref[1], preferred_element_type=jnp.float32,
                    precision=lax.Precision.HIGHEST),
        ],
        axis=1,
    )
    lane = lax.broadcasted_iota(jnp.int32, (T, E), 1)
    g1 = jnp.max(g, axis=1, keepdims=True)
    i1 = jnp.min(jnp.where(g == g1, lane, E), axis=1, keepdims=True)
    gm = jnp.where(lane == i1, -1e30, g)
    g2 = jnp.max(gm, axis=1, keepdims=True)
    i2 = jnp.min(jnp.where(gm == g2, lane, E), axis=1, keepdims=True)
    e2 = jnp.exp(g2 - g1)
    denom = 1.0 + e2
    w1 = 1.0 / denom
    w2 = e2 / denom
    for j in range(E_LOC):
        ej = ix * E_LOC + j
        w_ref[j] = jnp.where(i1 == ej, w1, 0.0) + jnp.where(i2 == ej, w2, 0.0)


def _ffn_body(xf_ref, w_ref, w1_ref, w2_ref, out_ref):
    e = pl.program_id(0)
    w_col = w_ref[0]
    sel_col = (w_col > 0.0).astype(jnp.float32)

    it = lax.broadcasted_iota(jnp.int32, (T, T), 0)
    jt = lax.broadcasted_iota(jnp.int32, (T, T), 1)
    m_tri = (it <= jt).astype(jnp.float32)
    ident = (it == jt).astype(jnp.float32)

    rank_row = lax.dot_general(
        sel_col, m_tri, (((0,), (0,)), ((), ())),
        preferred_element_type=jnp.float32, precision=lax.Precision.HIGHEST,
    )
    w_row = lax.dot_general(
        w_col, ident, (((0,), (0,)), ((), ())),
        preferred_element_type=jnp.float32, precision=lax.Precision.HIGHEST,
    )
    slot = rank_row.astype(jnp.int32) - 1
    sel_row = w_row > 0.0

    cap_iota = lax.broadcasted_iota(jnp.int32, (CAP, T), 0)
    p = jnp.where((cap_iota == slot) & sel_row, 1.0, 0.0)
    pw = p * w_row

    xsel = jnp.dot(p, xf_ref[:, :], preferred_element_type=jnp.float32)
    h = jnp.maximum(
        jnp.dot(xsel, w1_ref[0], preferred_element_type=jnp.float32), 0.0
    )
    o = jnp.dot(h, w2_ref[0], preferred_element_type=jnp.float32)
    contrib = lax.dot_general(
        pw, o, (((0,), (0,)), ((), ())), preferred_element_type=jnp.float32
    )

    @pl.when(e == 0)
    def _():
        out_ref[:, :] = contrib

    @pl.when(e != 0)
    def _():
        out_ref[:, :] = out_ref[:, :] + contrib


def _combine_body(of_ref, out_ref, comm_ref, send_sem, recv_sem):
    ix = lax.axis_index("x")
    iy = lax.axis_index("y")
    iz = lax.axis_index("z")
    partner = (1 - ix, iy, iz)

    barrier = pltpu.get_barrier_semaphore()
    pl.semaphore_signal(
        barrier, inc=1, device_id=partner, device_id_type=pl.DeviceIdType.MESH
    )
    pl.semaphore_wait(barrier, 1)

    pr = 1 - ix
    rdma = pltpu.make_async_remote_copy(
        src_ref=of_ref.at[pl.ds(pr * T_LOC, T_LOC)],
        dst_ref=comm_ref,
        send_sem=send_sem,
        recv_sem=recv_sem,
        device_id=partner,
        device_id_type=pl.DeviceIdType.MESH,
    )
    rdma.start()
    rdma.wait()

    out_ref[:, :] = of_ref[pl.ds(ix * T_LOC, T_LOC), :] + comm_ref[:, :]


def kernel(x, router, W1, W2):
    xf, w_e = pl.pallas_call(
        _exchange_body,
        out_shape=[
            jax.ShapeDtypeStruct((T, D), jnp.float32),
            jax.ShapeDtypeStruct((E_LOC, T, 1), jnp.float32),
        ],
        in_specs=[
            pl.BlockSpec(memory_space=pltpu.VMEM),
            pl.BlockSpec(memory_space=pltpu.VMEM),
        ],
        out_specs=[
            pl.BlockSpec(memory_space=pltpu.VMEM),
            pl.BlockSpec(memory_space=pltpu.VMEM),
        ],
        scratch_shapes=[
            pltpu.VMEM((2, D, E_LOC), jnp.float32),
            pltpu.SemaphoreType.DMA((2,)),
            pltpu.SemaphoreType.DMA((2,)),
        ],
        compiler_params=pltpu.CompilerParams(collective_id=0),
    )(x, router)

    of = pl.pallas_call(
        _ffn_body,
        grid=(E_LOC,),
        out_shape=jax.ShapeDtypeStruct((T, D), jnp.float32),
        in_specs=[
            pl.BlockSpec((T, D), lambda e: (0, 0)),
            pl.BlockSpec((1, T, 1), lambda e: (e, 0, 0)),
            pl.BlockSpec((1, D, F), lambda e: (e, 0, 0)),
            pl.BlockSpec((1, F, D), lambda e: (e, 0, 0)),
        ],
        out_specs=pl.BlockSpec((T, D), lambda e: (0, 0)),
        compiler_params=pltpu.CompilerParams(vmem_limit_bytes=100 * 1024 * 1024),
    )(xf, w_e, W1, W2)

    out = pl.pallas_call(
        _combine_body,
        out_shape=jax.ShapeDtypeStruct((T_LOC, D), jnp.float32),
        in_specs=[pl.BlockSpec(memory_space=pltpu.VMEM)],
        out_specs=pl.BlockSpec(memory_space=pltpu.VMEM),
        scratch_shapes=[
            pltpu.VMEM((T_LOC, D), jnp.float32),
            pltpu.SemaphoreType.DMA,
            pltpu.SemaphoreType.DMA,
        ],
        compiler_params=pltpu.CompilerParams(collective_id=1),
    )(of)

    return out
